# baseline (speedup 1.0000x reference)
import functools

import jax
import jax.numpy as jnp
from jax import lax
from jax.experimental import pallas as pl
from jax.experimental.pallas import tpu as pltpu
from jax.experimental.pallas import tpu_sc as plsc

_K = 16
_K2 = 32
_ROWS = 256
_N = 4096
_B = 2


def _knn_tile(q_ref, ptT_ref, idx_ref, dsel_ref):
    rr = q_ref.shape[1]
    n = ptT_ref.shape[2]

    x = ptT_ref[0, 0:1, :]
    y = ptT_ref[0, 1:2, :]
    z = ptT_ref[0, 2:3, :]
    qx = q_ref[0, :, 0:1]
    qy = q_ref[0, :, 1:2]
    qz = q_ref[0, :, 2:3]
    dx = x - qx
    dy = y - qy
    dz = z - qz
    d = (dx * dx + dy * dy) + dz * dz

    iota = lax.broadcasted_iota(jnp.int32, (rr, n), 1)
    lane2 = lax.broadcasted_iota(jnp.int32, (rr, _K2), 1)

    dsel = jnp.zeros((rr, _K2), jnp.float32)
    idxs = jnp.zeros((rr, _K2), jnp.int32)
    for t0 in range(0, _K2, 4):
        acc = None
        for u in range(4):
            du = d if acc is None else jnp.where(acc, jnp.inf, d)
            m = jnp.min(du, axis=1, keepdims=True)
            cand = jnp.where(du == m, iota, n)
            j = jnp.min(cand, axis=1, keepdims=True)
            acc = (iota == j) if acc is None else acc | (iota == j)
            hit = lane2 == (t0 + u)
            dsel = jnp.where(hit, m, dsel)
            idxs = jnp.where(hit, j, idxs)
        d = jnp.where(acc, jnp.inf, d)

    idx_ref[0] = idxs
    dsel_ref[0] = dsel


def _sqrt16(v):
    bits = lax.bitcast_convert_type(v, jnp.int32)
    seed = lax.bitcast_convert_type(
        (bits >> 1) + jnp.int32(0x1FBD1DF5), jnp.float32)
    t = seed
    for _ in range(4):
        t = 0.5 * (t + v / t)
    return t


def _rerank_sc(xyzT_ref, idx_ref, dsel_ref, out_ref, xv, yv, zv,
               idxv, dselv, outv):
    nc = 2
    wid = lax.axis_index("s") * nc + lax.axis_index("c")
    qbase = wid * (_B * _N // 32)
    b = lax.shift_right_logical(wid, 4)

    pltpu.sync_copy(xyzT_ref.at[pl.ds(b * 3 * _N, _N)], xv)
    pltpu.sync_copy(xyzT_ref.at[pl.ds((b * 3 + 1) * _N, _N)], yv)
    pltpu.sync_copy(xyzT_ref.at[pl.ds((b * 3 + 2) * _N, _N)], zv)
    pltpu.sync_copy(idx_ref.at[pl.ds(qbase * _K2, 256 * _K2)], idxv)
    pltpu.sync_copy(dsel_ref.at[pl.ds(qbase * _K2, 256 * _K2)], dselv)

    inv32 = jnp.float32(1.0 / _K2)

    def body(q, carry):
        i0 = idxv[pl.ds(q * _K2, 16)]
        i1 = idxv[pl.ds(q * _K2 + 16, 16)]
        d0 = dselv[pl.ds(q * _K2, 16)]
        d1 = dselv[pl.ds(q * _K2 + 16, 16)]
        x0 = plsc.load_gather(xv, [i0])
        x1 = plsc.load_gather(xv, [i1])
        y0 = plsc.load_gather(yv, [i0])
        y1 = plsc.load_gather(yv, [i1])
        z0 = plsc.load_gather(zv, [i0])
        z1 = plsc.load_gather(zv, [i1])

        ione = lax.iota(jnp.int32, 16)
        mx = jnp.full((16,), jnp.sum(x0 + x1), jnp.float32) * inv32
        my = jnp.full((16,), jnp.sum(y0 + y1), jnp.float32) * inv32
        mz = jnp.full((16,), jnp.sum(z0 + z1), jnp.float32) * inv32
        ax0 = x0 - mx
        ax1 = x1 - mx
        ay0 = y0 - my
        ay1 = y1 - my
        az0 = z0 - mz
        az1 = z1 - mz
        inv31 = jnp.float32(1.0 / (_K2 - 1))
        vx = jnp.full((16,), jnp.sum(ax0 * ax0 + ax1 * ax1), jnp.float32)
        vy = jnp.full((16,), jnp.sum(ay0 * ay0 + ay1 * ay1), jnp.float32)
        vz = jnp.full((16,), jnp.sum(az0 * az0 + az1 * az1), jnp.float32)
        var3 = jnp.where(ione < 1, vx, jnp.where(ione < 2, vy, vz)) * inv31
        std3 = _sqrt16(var3) + 1e-6
        dsum = jnp.sum(jnp.where(ione < 3, std3, 0.0))
        denom = jnp.full((16,), dsum, jnp.float32) / 3.0

        sq0 = (ax0 * ax0 + ay0 * ay0) + az0 * az0
        sq1 = (ax1 * ax1 + ay1 * ay1) + az1 * az1
        wd0 = d0 * jnp.exp(-sq0 / denom)
        wd1 = d1 * jnp.exp(-sq1 / denom)

        k0, v0 = plsc.sort_key_val(wd0, i0)
        k1, v1 = plsc.sort_key_val(wd1, i1)
        rk1 = lax.rev(k1, (0,))
        rv1 = lax.rev(v1, (0,))
        take = k0 <= rk1
        lok = jnp.where(take, k0, rk1)
        lov = jnp.where(take, v0, rv1)
        _, fv = plsc.sort_key_val(lok, lov)
        outv[pl.ds(q * _K, 16)] = fv
        return carry

    lax.fori_loop(0, 256, body, jnp.int32(0))
    pltpu.sync_copy(outv, out_ref.at[pl.ds(qbase * _K, 256 * _K)])


@functools.lru_cache(maxsize=1)
def _make_rerank_call():
    return functools.partial(
        pl.kernel,
        mesh=plsc.VectorSubcoreMesh(core_axis_name="c",
                                    subcore_axis_name="s"),
        compiler_params=pltpu.CompilerParams(needs_layout_passes=False),
        out_type=jax.ShapeDtypeStruct((_B * _N * _K,), jnp.int32),
        scratch_types=[
            pltpu.VMEM((_N,), jnp.float32),
            pltpu.VMEM((_N,), jnp.float32),
            pltpu.VMEM((_N,), jnp.float32),
            pltpu.VMEM((256 * _K2,), jnp.int32),
            pltpu.VMEM((256 * _K2,), jnp.float32),
            pltpu.VMEM((256 * _K,), jnp.int32),
        ],
    )(_rerank_sc)


def kernel(xyz):
    b, n, _ = xyz.shape
    xyzT = jnp.transpose(xyz, (0, 2, 1))
    idxs, dsel = pl.pallas_call(
        _knn_tile,
        grid=(b, n // _ROWS),
        in_specs=[
            pl.BlockSpec((1, _ROWS, 3), lambda bb, r: (bb, r, 0)),
            pl.BlockSpec((1, 3, n), lambda bb, r: (bb, 0, 0)),
        ],
        out_specs=[
            pl.BlockSpec((1, _ROWS, _K2), lambda bb, r: (bb, r, 0)),
            pl.BlockSpec((1, _ROWS, _K2), lambda bb, r: (bb, r, 0)),
        ],
        out_shape=[
            jax.ShapeDtypeStruct((b, n, _K2), jnp.int32),
            jax.ShapeDtypeStruct((b, n, _K2), jnp.float32),
        ],
    )(xyz, xyzT)
    out = _make_rerank_call()(xyzT.reshape(-1), idxs.reshape(-1),
                              dsel.reshape(-1))
    return out.reshape(b, n, _K)


# speedup vs baseline: 1.7329x; 1.7329x over previous
import functools

import jax
import jax.numpy as jnp
from jax import lax
from jax.experimental import pallas as pl
from jax.experimental.pallas import tpu as pltpu
from jax.experimental.pallas import tpu_sc as plsc

_K = 16
_K2 = 32
_ROWS = 256
_N = 4096
_B = 2


def _knn_tile(q_ref, ptT_ref, idx_ref):
    rr = q_ref.shape[1]
    n = ptT_ref.shape[2]

    x = ptT_ref[0, 0:1, :]
    y = ptT_ref[0, 1:2, :]
    z = ptT_ref[0, 2:3, :]
    qx = q_ref[0, :, 0:1]
    qy = q_ref[0, :, 1:2]
    qz = q_ref[0, :, 2:3]
    dx = x - qx
    dy = y - qy
    dz = z - qz
    d = (dx * dx + dy * dy) + dz * dz

    iota = lax.broadcasted_iota(jnp.int32, (rr, n), 1)
    lane2 = lax.broadcasted_iota(jnp.int32, (rr, _K2), 1)

    idxs = jnp.zeros((rr, _K2), jnp.int32)
    for t in range(_K2):
        j = jnp.argmin(d, axis=1, keepdims=True)
        d = jnp.where(iota == j, jnp.inf, d)
        idxs = jnp.where(lane2 == t, j, idxs)

    idx_ref[0] = idxs


def _sqrt16(v):
    bits = lax.bitcast_convert_type(v, jnp.int32)
    seed = lax.bitcast_convert_type(
        (bits >> 1) + jnp.int32(0x1FBD1DF5), jnp.float32)
    t = seed
    for _ in range(4):
        t = 0.5 * (t + v / t)
    return t


def _rerank_sc(xyzT_ref, idx_ref, out_ref, xv, yv, zv, idxv, outv):
    nc = 2
    wid = lax.axis_index("s") * nc + lax.axis_index("c")
    qbase = wid * (_B * _N // 32)
    b = lax.shift_right_logical(wid, 4)
    qloc0 = (wid & 15) * 256

    pltpu.sync_copy(xyzT_ref.at[pl.ds(b * 3 * _N, _N)], xv)
    pltpu.sync_copy(xyzT_ref.at[pl.ds((b * 3 + 1) * _N, _N)], yv)
    pltpu.sync_copy(xyzT_ref.at[pl.ds((b * 3 + 2) * _N, _N)], zv)
    pltpu.sync_copy(idx_ref.at[pl.ds(qbase * _K2, 256 * _K2)], idxv)

    inv32 = jnp.float32(1.0 / _K2)

    def body(q, carry):
        i0 = idxv[pl.ds(q * _K2, 16)]
        i1 = idxv[pl.ds(q * _K2 + 16, 16)]
        qsplat = jnp.full((16,), qloc0 + q, jnp.int32)
        qx = plsc.load_gather(xv, [qsplat])
        qy = plsc.load_gather(yv, [qsplat])
        qz = plsc.load_gather(zv, [qsplat])
        x0 = plsc.load_gather(xv, [i0])
        x1 = plsc.load_gather(xv, [i1])
        y0 = plsc.load_gather(yv, [i0])
        y1 = plsc.load_gather(yv, [i1])
        z0 = plsc.load_gather(zv, [i0])
        z1 = plsc.load_gather(zv, [i1])

        dx0 = x0 - qx
        dy0 = y0 - qy
        dz0 = z0 - qz
        dx1 = x1 - qx
        dy1 = y1 - qy
        dz1 = z1 - qz
        d0 = (dx0 * dx0 + dy0 * dy0) + dz0 * dz0
        d1 = (dx1 * dx1 + dy1 * dy1) + dz1 * dz1

        ione = lax.iota(jnp.int32, 16)
        mx = jnp.full((16,), jnp.sum(x0 + x1), jnp.float32) * inv32
        my = jnp.full((16,), jnp.sum(y0 + y1), jnp.float32) * inv32
        mz = jnp.full((16,), jnp.sum(z0 + z1), jnp.float32) * inv32
        ax0 = x0 - mx
        ax1 = x1 - mx
        ay0 = y0 - my
        ay1 = y1 - my
        az0 = z0 - mz
        az1 = z1 - mz
        inv31 = jnp.float32(1.0 / (_K2 - 1))
        vx = jnp.full((16,), jnp.sum(ax0 * ax0 + ax1 * ax1), jnp.float32)
        vy = jnp.full((16,), jnp.sum(ay0 * ay0 + ay1 * ay1), jnp.float32)
        vz = jnp.full((16,), jnp.sum(az0 * az0 + az1 * az1), jnp.float32)
        var3 = jnp.where(ione < 1, vx, jnp.where(ione < 2, vy, vz)) * inv31
        std3 = _sqrt16(var3) + 1e-6
        dsum = jnp.sum(jnp.where(ione < 3, std3, 0.0))
        denom = jnp.full((16,), dsum, jnp.float32) / 3.0

        sq0 = (ax0 * ax0 + ay0 * ay0) + az0 * az0
        sq1 = (ax1 * ax1 + ay1 * ay1) + az1 * az1
        wd0 = d0 * jnp.exp(-sq0 / denom)
        wd1 = d1 * jnp.exp(-sq1 / denom)

        k0, v0 = plsc.sort_key_val(wd0, i0)
        k1, v1 = plsc.sort_key_val(wd1, i1)
        rk1 = lax.rev(k1, (0,))
        rv1 = lax.rev(v1, (0,))
        take = k0 <= rk1
        lok = jnp.where(take, k0, rk1)
        lov = jnp.where(take, v0, rv1)
        _, fv = plsc.sort_key_val(lok, lov)
        outv[pl.ds(q * _K, 16)] = fv
        return carry

    lax.fori_loop(0, 256, body, jnp.int32(0))
    pltpu.sync_copy(outv, out_ref.at[pl.ds(qbase * _K, 256 * _K)])


@functools.lru_cache(maxsize=1)
def _make_rerank_call():
    return functools.partial(
        pl.kernel,
        mesh=plsc.VectorSubcoreMesh(core_axis_name="c",
                                    subcore_axis_name="s"),
        compiler_params=pltpu.CompilerParams(needs_layout_passes=False),
        out_type=jax.ShapeDtypeStruct((_B * _N * _K,), jnp.int32),
        scratch_types=[
            pltpu.VMEM((_N,), jnp.float32),
            pltpu.VMEM((_N,), jnp.float32),
            pltpu.VMEM((_N,), jnp.float32),
            pltpu.VMEM((256 * _K2,), jnp.int32),
            pltpu.VMEM((256 * _K,), jnp.int32),
        ],
    )(_rerank_sc)


def kernel(xyz):
    b, n, _ = xyz.shape
    xyzT = jnp.transpose(xyz, (0, 2, 1))
    idxs = pl.pallas_call(
        _knn_tile,
        grid=(b, n // _ROWS),
        in_specs=[
            pl.BlockSpec((1, _ROWS, 3), lambda bb, r: (bb, r, 0)),
            pl.BlockSpec((1, 3, n), lambda bb, r: (bb, 0, 0)),
        ],
        out_specs=pl.BlockSpec((1, _ROWS, _K2), lambda bb, r: (bb, r, 0)),
        out_shape=jax.ShapeDtypeStruct((b, n, _K2), jnp.int32),
    )(xyz, xyzT)
    out = _make_rerank_call()(xyzT.reshape(-1), idxs.reshape(-1))
    return out.reshape(b, n, _K)
